# baseline (speedup 1.0000x reference)
import functools

import jax
import jax.numpy as jnp
from jax import lax
from jax.experimental import pallas as pl
from jax.experimental.pallas import tpu as pltpu
from jax.experimental.pallas import tpu_sc as plsc

L = 2048
D = 768
HID = 2 * D
TOPK = 8
NG = TOPK + 1
KSTEPS = 5
LAM = 0.01
E_TRUE = (L - 1) + TOPK * L
NE = NG * L

BF = jnp.bfloat16
F32 = jnp.float32


def _topk_body(attn_ref, idx_ref, w_ref, pij_ref, pji_ref, wsum_ref):
    i = pl.program_id(0)
    rblk = attn_ref.shape[0]
    a = attn_ref[...]
    rows = i * rblk + lax.broadcasted_iota(jnp.int32, (rblk, L), 0)
    cols = lax.broadcasted_iota(jnp.int32, (rblk, L), 1)
    m = jnp.where(cols <= rows - 2, a, -10000.0)
    vals, idxs = [], []
    for _ in range(TOPK):
        mx = jnp.max(m, axis=1, keepdims=True)
        ismax = m == mx
        ik = jnp.min(jnp.where(ismax, cols, L), axis=1, keepdims=True)
        m = jnp.where(cols == ik, -jnp.float32(jnp.inf), m)
        vals.append(mx)
        idxs.append(ik)
    vals = jnp.concatenate(vals, axis=1)
    idx = jnp.concatenate(idxs, axis=1)
    mxv = jnp.max(vals, axis=1, keepdims=True)
    ex = jnp.exp(vals - mxv)
    w8 = ex / jnp.sum(ex, axis=1, keepdims=True)
    rows8 = i * rblk + lax.broadcasted_iota(jnp.int32, (rblk, TOPK), 0)
    rel = idx - rows8
    idx_ref[...] = idx
    w_ref[...] = w8
    pij_ref[...] = jnp.clip(rel + 1024, 0, 2047)
    pji_ref[...] = jnp.clip(-rel + 1024, 0, 2047)

    @pl.when(i == 0)
    def _():
        wsum_ref[0, 0] = 0.0

    wsum_ref[0, 0] += jnp.sum(w8)


def _topk_call(a2):
    rblk = 256
    grid = L // rblk
    return pl.pallas_call(
        _topk_body,
        grid=(grid,),
        in_specs=[pl.BlockSpec((rblk, L), lambda i: (i, 0))],
        out_specs=[
            pl.BlockSpec((rblk, TOPK), lambda i: (i, 0)),
            pl.BlockSpec((rblk, TOPK), lambda i: (i, 0)),
            pl.BlockSpec((rblk, TOPK), lambda i: (i, 0)),
            pl.BlockSpec((rblk, TOPK), lambda i: (i, 0)),
            pl.BlockSpec(memory_space=pltpu.SMEM),
        ],
        out_shape=[
            jax.ShapeDtypeStruct((L, TOPK), jnp.int32),
            jax.ShapeDtypeStruct((L, TOPK), F32),
            jax.ShapeDtypeStruct((L, TOPK), jnp.int32),
            jax.ShapeDtypeStruct((L, TOPK), jnp.int32),
            jax.ShapeDtypeStruct((1, 1), F32),
        ],
    )(a2)


def _pre_body(h_ref, wa_ref, wb_ref, ha_ref, hb_ref):
    x = h_ref[...]
    ha_ref[...] = jnp.dot(x, wa_ref[...], preferred_element_type=F32)
    hb_ref[...] = jnp.dot(x, wb_ref[...], preferred_element_type=F32)


def _pre_call(h_bf, wa_bf, wb_bf):
    cblk = 512
    grid = HID // cblk
    return pl.pallas_call(
        _pre_body,
        grid=(grid,),
        in_specs=[
            pl.BlockSpec((L, D), lambda i: (0, 0)),
            pl.BlockSpec((D, cblk), lambda i: (0, i)),
            pl.BlockSpec((D, cblk), lambda i: (0, i)),
        ],
        out_specs=[
            pl.BlockSpec((L, cblk), lambda i: (0, i)),
            pl.BlockSpec((L, cblk), lambda i: (0, i)),
        ],
        out_shape=[
            jax.ShapeDtypeStruct((L, HID), F32),
            jax.ShapeDtypeStruct((L, HID), F32),
        ],
    )(h_bf, wa_bf, wb_bf)


def _ln_relu(x, g1, be1):
    mu = jnp.mean(x, axis=1, keepdims=True)
    xc = x - mu
    var = jnp.mean(xc * xc, axis=1, keepdims=True)
    ln = xc / jnp.sqrt(var + 1e-5) * g1 + be1
    return jnp.maximum(ln, 0.0)


def _mlp_body(ha_ref, hb_ref, ght_ref, gpij_ref, gpji_ref,
              wa_ref, wb_ref, wc_ref, w2_ref,
              b1_ref, g1_ref, be1_ref, b2_ref,
              v1_ref, v2_ref):
    ght = ght_ref[0].astype(BF)
    b1 = b1_ref[...]
    g1 = g1_ref[...]
    be1 = be1_ref[...]
    b2 = b2_ref[...]
    w2 = w2_ref[...]
    pos_ij = jnp.dot(gpij_ref[0].astype(BF), wc_ref[...],
                     preferred_element_type=F32)
    pos_ji = jnp.dot(gpji_ref[0].astype(BF), wc_ref[...],
                     preferred_element_type=F32)
    pre_ij = (ha_ref[...] + pos_ij + b1
              + jnp.dot(ght, wb_ref[...], preferred_element_type=F32))
    pre_ji = (hb_ref[...] + pos_ji + b1
              + jnp.dot(ght, wa_ref[...], preferred_element_type=F32))
    for pre, out in ((pre_ij, v1_ref), (pre_ji, v2_ref)):
        hdn = _ln_relu(pre, g1, be1).astype(BF)
        v = jnp.dot(hdn, w2, preferred_element_type=F32) + b2
        n = jnp.sqrt(jnp.sum(v * v, axis=1, keepdims=True))
        out[0] = (v / jnp.maximum(n, 1e-8)).astype(BF)


def _mlp_call(ha, hb, ght, gpij, gpji, wa_bf, wb_bf, wc_bf, w2_bf,
              b1, g1, be1, b2):
    jblk = 256
    grid = (L // jblk, NG)
    row = lambda j, k: (j, 0)
    grp = lambda j, k: (k, j, 0)
    fixed = lambda j, k: (0, 0)
    return pl.pallas_call(
        _mlp_body,
        grid=grid,
        in_specs=[
            pl.BlockSpec((jblk, HID), row),
            pl.BlockSpec((jblk, HID), row),
            pl.BlockSpec((1, jblk, D), grp),
            pl.BlockSpec((1, jblk, D), grp),
            pl.BlockSpec((1, jblk, D), grp),
            pl.BlockSpec((D, HID), fixed),
            pl.BlockSpec((D, HID), fixed),
            pl.BlockSpec((D, HID), fixed),
            pl.BlockSpec((HID, D), fixed),
            pl.BlockSpec((1, HID), fixed),
            pl.BlockSpec((1, HID), fixed),
            pl.BlockSpec((1, HID), fixed),
            pl.BlockSpec((1, D), fixed),
        ],
        out_specs=[
            pl.BlockSpec((1, jblk, D), grp),
            pl.BlockSpec((1, jblk, D), grp),
        ],
        out_shape=[
            jax.ShapeDtypeStruct((NG, L, D), BF),
            jax.ShapeDtypeStruct((NG, L, D), BF),
        ],
    )(ha, hb, ght, gpij, gpji, wa_bf, wb_bf, wc_bf, w2_bf, b1, g1, be1, b2)


def _step_body(hc_ref, gt_ref, v1_ref, v2_ref, w_ref, wsum_ref,
               asum_ref, contrib_ref):
    hcb = hc_ref[...]
    winv = 1.0 / (2047.0 + wsum_ref[0, 0] + 1e-8)
    acc = jnp.zeros_like(hcb)
    for k in range(NG):
        gtk = gt_ref[k].astype(F32)
        v1k = v1_ref[k].astype(F32)
        v2k = v2_ref[k].astype(F32)
        wk = w_ref[:, k:k + 1] * winv
        dk = jnp.sum(v2k * gtk, axis=1, keepdims=True)
        t1 = gtk - 2.0 * dk * v2k
        bq = jnp.sum(v1k * t1, axis=1, keepdims=True)
        q_gt = t1 - 2.0 * bq * v1k
        ak = jnp.sum(v1k * hcb, axis=1, keepdims=True)
        t2 = hcb - 2.0 * ak * v1k
        cq = jnp.sum(v2k * t2, axis=1, keepdims=True)
        qt_h = t2 - 2.0 * cq * v2k
        acc = acc + wk * (hcb - q_gt)
        contrib_ref[k] = wk * (gtk - qt_h)
    asum_ref[...] = acc


def _step_call(hc, gt3, v1, v2, w2d, wsum):
    jblk = 128
    grid = (L // jblk,)
    return pl.pallas_call(
        _step_body,
        grid=grid,
        in_specs=[
            pl.BlockSpec((jblk, D), lambda i: (i, 0)),
            pl.BlockSpec((NG, jblk, D), lambda i: (0, i, 0)),
            pl.BlockSpec((NG, jblk, D), lambda i: (0, i, 0)),
            pl.BlockSpec((NG, jblk, D), lambda i: (0, i, 0)),
            pl.BlockSpec((jblk, 16), lambda i: (i, 0)),
            pl.BlockSpec(memory_space=pltpu.SMEM),
        ],
        out_specs=[
            pl.BlockSpec((jblk, D), lambda i: (i, 0)),
            pl.BlockSpec((NG, jblk, D), lambda i: (0, i, 0)),
        ],
        out_shape=[
            jax.ShapeDtypeStruct((L, D), F32),
            jax.ShapeDtypeStruct((NG, L, D), F32),
        ],
    )(hc, gt3, v1, v2, w2d, wsum)


def _update_body(hc_ref, h0_ref, asum_ref, p_ref, eta_ref, out_ref, outb_ref):
    g = asum_ref[...] + p_ref[0] + p_ref[1] + LAM * (hc_ref[...] - h0_ref[...])
    hn = hc_ref[...] - jnp.abs(eta_ref[0, 0]) * g
    out_ref[...] = hn
    outb_ref[...] = hn.astype(BF)


def _update_call(hc, h0, asum, p, eta11):
    jblk = 256
    return pl.pallas_call(
        _update_body,
        grid=(L // jblk,),
        in_specs=[
            pl.BlockSpec((jblk, D), lambda i: (i, 0)),
            pl.BlockSpec((jblk, D), lambda i: (i, 0)),
            pl.BlockSpec((jblk, D), lambda i: (i, 0)),
            pl.BlockSpec((2, jblk, D), lambda i: (0, i, 0)),
            pl.BlockSpec(memory_space=pltpu.SMEM),
        ],
        out_specs=[
            pl.BlockSpec((jblk, D), lambda i: (i, 0)),
            pl.BlockSpec((jblk, D), lambda i: (i, 0)),
        ],
        out_shape=[
            jax.ShapeDtypeStruct((L, D), F32),
            jax.ShapeDtypeStruct((L, D), BF),
        ],
    )(hc, h0, asum, p, eta11)


def _energy_body(hc_ref, gt_ref, v1_ref, v2_ref, w_ref, wsum_ref, out_ref):
    i = pl.program_id(0)
    hcb = hc_ref[...]
    winv = 1.0 / (2047.0 + wsum_ref[0, 0] + 1e-8)
    nj = jnp.sum(hcb * hcb, axis=1, keepdims=True)
    tot = jnp.zeros((1, 1), F32)
    for k in range(NG):
        gtk = gt_ref[k].astype(F32)
        v1k = v1_ref[k].astype(F32)
        v2k = v2_ref[k].astype(F32)
        wk = w_ref[:, k:k + 1] * winv
        dk = jnp.sum(v2k * gtk, axis=1, keepdims=True)
        t1 = gtk - 2.0 * dk * v2k
        bq = jnp.sum(v1k * t1, axis=1, keepdims=True)
        q_gt = t1 - 2.0 * bq * v1k
        hj_qgt = jnp.sum(hcb * q_gt, axis=1, keepdims=True)
        nt = jnp.sum(gtk * gtk, axis=1, keepdims=True)
        tot = tot + jnp.sum(wk * (nj + nt - 2.0 * hj_qgt),
                            axis=0, keepdims=True)

    @pl.when(i == 0)
    def _():
        out_ref[0, 0] = 0.0

    out_ref[0, 0] += tot[0, 0] * (50.0 / (E_TRUE * D))


def _energy_call(hc, gt3, v1, v2, w2d, wsum):
    jblk = 128
    return pl.pallas_call(
        _energy_body,
        grid=(L // jblk,),
        in_specs=[
            pl.BlockSpec((jblk, D), lambda i: (i, 0)),
            pl.BlockSpec((NG, jblk, D), lambda i: (0, i, 0)),
            pl.BlockSpec((NG, jblk, D), lambda i: (0, i, 0)),
            pl.BlockSpec((NG, jblk, D), lambda i: (0, i, 0)),
            pl.BlockSpec((jblk, 16), lambda i: (i, 0)),
            pl.BlockSpec(memory_space=pltpu.SMEM),
        ],
        out_specs=pl.BlockSpec(memory_space=pltpu.SMEM),
        out_shape=jax.ShapeDtypeStruct((1, 1), F32),
    )(hc, gt3, v1, v2, w2d, wsum)


def _sc_gather(table, gidx, chunk=32, nbuf=4):
    n_rows = gidx.shape[0]
    width = table.shape[1]
    dt = table.dtype
    nw = 32
    per_w = n_rows // nw
    n_chunks = per_w // chunk
    mesh = plsc.VectorSubcoreMesh(core_axis_name="c", subcore_axis_name="s")

    @functools.partial(
        pl.kernel,
        mesh=mesh,
        out_type=jax.ShapeDtypeStruct((n_rows, width), dt),
        scratch_types=(
            [pltpu.VMEM((per_w,), jnp.int32)]
            + [pltpu.VMEM((chunk, width), dt) for _ in range(nbuf)]
            + [pltpu.SemaphoreType.DMA, pltpu.SemaphoreType.DMA]
        ),
    )
    def k(table_hbm, idx_hbm, out_hbm, idx_v, *bufs_sems):
        bufs = bufs_sems[:nbuf]
        semg, semo = bufs_sems[nbuf], bufs_sems[nbuf + 1]
        wid = lax.axis_index("s") * 2 + lax.axis_index("c")
        base = wid * per_w
        pltpu.async_copy(idx_hbm.at[pl.ds(base, per_w)], idx_v, semg).wait()

        @pl.loop(0, n_chunks, step=nbuf)
        def _(ci):
            for b in range(nbuf):
                cj = ci + b

                @pl.when(cj < n_chunks)
                def _():
                    off = base + cj * chunk

                    @pl.when(cj >= nbuf)
                    def _():
                        pltpu.make_async_copy(
                            bufs[b], out_hbm.at[pl.ds(off, chunk)], semo
                        ).wait()

                    pltpu.async_copy(
                        table_hbm.at[idx_v.at[pl.ds(cj * chunk, chunk)]],
                        bufs[b], semg).wait()
                    pltpu.async_copy(bufs[b], out_hbm.at[pl.ds(off, chunk)],
                                     semo)

        for _ in range(min(nbuf, n_chunks)):
            pltpu.make_async_copy(bufs[0], out_hbm.at[pl.ds(base, chunk)],
                                  semo).wait()

    return k(table, gidx)


def _sc_scatter(src, sidx_rep):
    n_rows = sidx_rep.shape[0]
    nsub = 16
    cw = D // nsub
    per_core = n_rows // 2
    chunk = 192
    n_chunks = per_core // chunk
    assert n_chunks % 2 == 0
    mesh = plsc.VectorSubcoreMesh(core_axis_name="c", subcore_axis_name="s")

    @functools.partial(
        pl.kernel,
        mesh=mesh,
        out_type=jax.ShapeDtypeStruct((2, L, D), F32),
        compiler_params=pltpu.CompilerParams(use_tc_tiling_on_sc=False,
                                             needs_layout_passes=False),
        scratch_types=(
            [pltpu.VMEM((chunk, 16), jnp.int32) for _ in range(2)]
            + [pltpu.VMEM((chunk, cw), F32) for _ in range(2)]
            + [pltpu.VMEM((L, cw), F32), pltpu.SemaphoreType.DMA]
        ),
    )
    def k(src_hbm, idx_hbm, out_hbm, idx0, idx1, rows0, rows1, acc_v, semd):
        idxb = (idx0, idx1)
        rowsb = (rows0, rows1)
        core = lax.axis_index("c")
        sid = lax.axis_index("s")
        cbase = sid * cw
        ebase = core * per_core
        col16 = lax.iota(jnp.int32, 16)
        z16 = jnp.zeros((16,), F32)

        @pl.loop(0, L)
        def _(r):
            for c16 in range(0, cw, 16):
                acc_v[r, pl.ds(c16, 16)] = z16

        def fire(cj, b):
            off = ebase + cj * chunk
            pltpu.async_copy(idx_hbm.at[pl.ds(off, chunk)], idxb[b], semd)
            pltpu.async_copy(
                src_hbm.at[pl.ds(off, chunk), pl.ds(cbase, cw)],
                rowsb[b], semd)

        fire(0, 0)
        fire(1, 1)

        @pl.loop(0, n_chunks, step=2)
        def _(ci):
            for b in range(2):
                cj = ci + b

                @pl.when(cj + 2 < n_chunks)
                def _():
                    fire(cj + 2, b)

                pltpu.make_async_copy(idx_hbm.at[pl.ds(ebase, chunk)],
                                      idxb[b], semd).wait()
                pltpu.make_async_copy(
                    src_hbm.at[pl.ds(ebase, chunk), pl.ds(cbase, cw)],
                    rowsb[b], semd).wait()

                @pl.loop(0, chunk)
                def _(r):
                    trow = idxb[b][r]
                    for c16 in range(0, cw, 16):
                        x = rowsb[b][r, pl.ds(c16, 16)]
                        plsc.addupdate_scatter(acc_v, [trow, col16 + c16], x)

        pltpu.sync_copy(acc_v,
                        out_hbm.at[core, pl.ds(0, L), pl.ds(cbase, cw)])

    return k(src, sidx_rep)


def kernel(h, attn, pos_emb, W1, b1, g1, be1, W2, b2, eta):
    h2 = h[0]
    a2 = attn[0]
    idx8, w8, pij8, pji8, wsum = _topk_call(a2)

    j = jnp.arange(L, dtype=jnp.int32)
    tgt = jnp.concatenate([jnp.maximum(j - 1, 0)[None, :], idx8.T], axis=0)
    pijg = jnp.concatenate(
        [jnp.full((1, L), 1023, jnp.int32), pij8.T], axis=0)
    pjig = jnp.concatenate(
        [jnp.full((1, L), 1025, jnp.int32), pji8.T], axis=0)
    w2d = jnp.concatenate(
        [(j >= 1).astype(F32)[:, None], w8, jnp.zeros((L, 16 - 1 - TOPK), F32)],
        axis=1)
    tgt_flat = tgt.reshape(-1)
    gidx = jnp.concatenate(
        [tgt_flat, pijg.reshape(-1) + L, pjig.reshape(-1) + L])

    tbl_bf = jnp.concatenate([h2, pos_emb], axis=0).astype(BF)
    tbl_i = lax.bitcast_convert_type(
        tbl_bf.reshape(2 * L, D // 2, 2), jnp.int32)
    g_i = _sc_gather(tbl_i, gidx, chunk=64)
    g_all = lax.bitcast_convert_type(g_i, BF).reshape(3 * NE, D)
    ght = g_all[0:NE].reshape(NG, L, D)
    gpij = g_all[NE:2 * NE].reshape(NG, L, D)
    gpji = g_all[2 * NE:].reshape(NG, L, D)

    w1a_bf = W1[:D].astype(BF)
    w1b_bf = W1[D:2 * D].astype(BF)
    w1c_bf = W1[2 * D:].astype(BF)
    ha, hb = _pre_call(h2.astype(BF), w1a_bf, w1b_bf)
    v1, v2 = _mlp_call(
        ha, hb, ght, gpij, gpji, w1a_bf, w1b_bf, w1c_bf, W2.astype(BF),
        b1.reshape(1, HID), g1.reshape(1, HID), be1.reshape(1, HID),
        b2.reshape(1, D))

    eta11 = eta.reshape(1, 1)
    tgt_rep = jnp.broadcast_to(tgt_flat[:, None], (NE, 16))

    def pack_bf(x):
        return lax.bitcast_convert_type(
            x.reshape(x.shape[0], D // 2, 2), jnp.int32)

    def unpack_bf(x):
        return lax.bitcast_convert_type(x, BF).reshape(x.shape[0], D)

    hc = h2
    hcb = h2.astype(BF)
    for _ in range(KSTEPS):
        gt_i = _sc_gather(pack_bf(hcb), tgt_flat, chunk=64)
        gt3 = unpack_bf(gt_i).reshape(NG, L, D)
        asum, contrib = _step_call(hc, gt3, v1, v2, w2d, wsum)
        p = _sc_scatter(contrib.reshape(NE, D), tgt_rep)
        hc, hcb = _update_call(hc, h2, asum, p, eta11)

    gt_i = _sc_gather(pack_bf(hcb), tgt_flat, chunk=64)
    gt3 = unpack_bf(gt_i).reshape(NG, L, D)
    energy = _energy_call(hc, gt3, v1, v2, w2d, wsum)
    return hc[None], energy.reshape(())


# speedup vs baseline: 1.9748x; 1.9748x over previous
import functools

import jax
import jax.numpy as jnp
import numpy as np
from jax import lax
from jax.experimental import pallas as pl
from jax.experimental.pallas import tpu as pltpu
from jax.experimental.pallas import tpu_sc as plsc

L = 2048
D = 768
HID = 2 * D
TOPK = 8
NG = TOPK + 1
KSTEPS = 5
LAM = 0.01
E_TRUE = (L - 1) + TOPK * L
NE = NG * L

BF = jnp.bfloat16
F32 = jnp.float32


def _topk_body(attn_ref, idx_ref, w_ref, pij_ref, pji_ref, wsum_ref):
    i = pl.program_id(0)
    rblk = attn_ref.shape[0]
    a = attn_ref[...]
    rows = i * rblk + lax.broadcasted_iota(jnp.int32, (rblk, L), 0)
    cols = lax.broadcasted_iota(jnp.int32, (rblk, L), 1)
    m = jnp.where(cols <= rows - 2, a, -10000.0)
    vals, idxs = [], []
    for _ in range(TOPK):
        mx = jnp.max(m, axis=1, keepdims=True)
        ismax = m == mx
        ik = jnp.min(jnp.where(ismax, cols, L), axis=1, keepdims=True)
        m = jnp.where(cols == ik, -jnp.float32(jnp.inf), m)
        vals.append(mx)
        idxs.append(ik)
    vals = jnp.concatenate(vals, axis=1)
    idx = jnp.concatenate(idxs, axis=1)
    mxv = jnp.max(vals, axis=1, keepdims=True)
    ex = jnp.exp(vals - mxv)
    w8 = ex / jnp.sum(ex, axis=1, keepdims=True)
    rows8 = i * rblk + lax.broadcasted_iota(jnp.int32, (rblk, TOPK), 0)
    rel = idx - rows8
    idx_ref[...] = idx
    w_ref[...] = w8
    pij_ref[...] = jnp.clip(rel + 1024, 0, 2047)
    pji_ref[...] = jnp.clip(-rel + 1024, 0, 2047)

    @pl.when(i == 0)
    def _():
        wsum_ref[0, 0] = 0.0

    wsum_ref[0, 0] += jnp.sum(w8)


def _topk_call(a2):
    rblk = 256
    grid = L // rblk
    return pl.pallas_call(
        _topk_body,
        grid=(grid,),
        in_specs=[pl.BlockSpec((rblk, L), lambda i: (i, 0))],
        out_specs=[
            pl.BlockSpec((rblk, TOPK), lambda i: (i, 0)),
            pl.BlockSpec((rblk, TOPK), lambda i: (i, 0)),
            pl.BlockSpec((rblk, TOPK), lambda i: (i, 0)),
            pl.BlockSpec((rblk, TOPK), lambda i: (i, 0)),
            pl.BlockSpec(memory_space=pltpu.SMEM),
        ],
        out_shape=[
            jax.ShapeDtypeStruct((L, TOPK), jnp.int32),
            jax.ShapeDtypeStruct((L, TOPK), F32),
            jax.ShapeDtypeStruct((L, TOPK), jnp.int32),
            jax.ShapeDtypeStruct((L, TOPK), jnp.int32),
            jax.ShapeDtypeStruct((1, 1), F32),
        ],
    )(a2)


def _pre_body(h_ref, wa_ref, wb_ref, ha_ref, hb_ref):
    x = h_ref[...]
    ha_ref[...] = jnp.dot(x, wa_ref[...], preferred_element_type=F32)
    hb_ref[...] = jnp.dot(x, wb_ref[...], preferred_element_type=F32)


def _pre_call(h_bf, wa_bf, wb_bf):
    cblk = 512
    grid = HID // cblk
    return pl.pallas_call(
        _pre_body,
        grid=(grid,),
        in_specs=[
            pl.BlockSpec((L, D), lambda i: (0, 0)),
            pl.BlockSpec((D, cblk), lambda i: (0, i)),
            pl.BlockSpec((D, cblk), lambda i: (0, i)),
        ],
        out_specs=[
            pl.BlockSpec((L, cblk), lambda i: (0, i)),
            pl.BlockSpec((L, cblk), lambda i: (0, i)),
        ],
        out_shape=[
            jax.ShapeDtypeStruct((L, HID), F32),
            jax.ShapeDtypeStruct((L, HID), F32),
        ],
    )(h_bf, wa_bf, wb_bf)


def _ln_relu(x, g1, be1):
    mu = jnp.mean(x, axis=1, keepdims=True)
    xc = x - mu
    var = jnp.mean(xc * xc, axis=1, keepdims=True)
    ln = xc / jnp.sqrt(var + 1e-5) * g1 + be1
    return jnp.maximum(ln, 0.0)


def _mlp_body(ha_ref, hb_ref, ght_ref, gpij_ref, gpji_ref,
              wa_ref, wb_ref, wc_ref, w2_ref,
              b1_ref, g1_ref, be1_ref, b2_ref,
              v1_ref, v2_ref):
    ght = _unpack_cols(ght_ref[0]).astype(BF)
    b1 = b1_ref[...]
    g1 = g1_ref[...]
    be1 = be1_ref[...]
    b2 = b2_ref[...]
    w2 = w2_ref[...]
    pos_ij = jnp.dot(_unpack_cols(gpij_ref[0]).astype(BF), wc_ref[...],
                     preferred_element_type=F32)
    pos_ji = jnp.dot(_unpack_cols(gpji_ref[0]).astype(BF), wc_ref[...],
                     preferred_element_type=F32)
    pre_ij = (ha_ref[...] + pos_ij + b1
              + jnp.dot(ght, wb_ref[...], preferred_element_type=F32))
    pre_ji = (hb_ref[...] + pos_ji + b1
              + jnp.dot(ght, wa_ref[...], preferred_element_type=F32))
    for pre, out in ((pre_ij, v1_ref), (pre_ji, v2_ref)):
        hdn = _ln_relu(pre, g1, be1).astype(BF)
        v = jnp.dot(hdn, w2, preferred_element_type=F32) + b2
        n = jnp.sqrt(jnp.sum(v * v, axis=1, keepdims=True))
        out[0] = (v / jnp.maximum(n, 1e-8)).astype(BF)


def _mlp_call(ha, hb, ght, gpij, gpji, wa_bf, wb_bf, wc_bf, w2_bf,
              b1, g1, be1, b2):
    jblk = 256
    grid = (L // jblk, NG)
    row = lambda j, k: (j, 0)
    grp = lambda j, k: (k, j, 0)
    fixed = lambda j, k: (0, 0)
    return pl.pallas_call(
        _mlp_body,
        grid=grid,
        in_specs=[
            pl.BlockSpec((jblk, HID), row),
            pl.BlockSpec((jblk, HID), row),
            pl.BlockSpec((1, jblk, HD), grp),
            pl.BlockSpec((1, jblk, HD), grp),
            pl.BlockSpec((1, jblk, HD), grp),
            pl.BlockSpec((D, HID), fixed),
            pl.BlockSpec((D, HID), fixed),
            pl.BlockSpec((D, HID), fixed),
            pl.BlockSpec((HID, D), fixed),
            pl.BlockSpec((1, HID), fixed),
            pl.BlockSpec((1, HID), fixed),
            pl.BlockSpec((1, HID), fixed),
            pl.BlockSpec((1, D), fixed),
        ],
        out_specs=[
            pl.BlockSpec((1, jblk, D), grp),
            pl.BlockSpec((1, jblk, D), grp),
        ],
        out_shape=[
            jax.ShapeDtypeStruct((NG, L, D), BF),
            jax.ShapeDtypeStruct((NG, L, D), BF),
        ],
    )(ha, hb, ght, gpij, gpji, wa_bf, wb_bf, wc_bf, w2_bf, b1, g1, be1, b2)


HD = D // 2


def _unpack_cols(w):
    e = lax.bitcast_convert_type(w << 16, F32)
    o = lax.bitcast_convert_type(w & jnp.int32(-65536), F32)
    return jnp.concatenate([e, o], axis=1)


def _pack_cols(x):
    xb = lax.bitcast_convert_type(x, jnp.int32)
    r = xb + 0x7FFF + ((xb >> 16) & 1)
    b16 = (r >> 16) & 0xFFFF
    return b16[:, :HD] | (b16[:, HD:] << 16)


def _step_body(hc_ref, gt_ref, v1_ref, v2_ref, w_ref, wsum_ref,
               asum_ref, contrib_ref):
    hcb = hc_ref[...]
    winv = 1.0 / (2047.0 + wsum_ref[0, 0] + 1e-8)
    acc = jnp.zeros_like(hcb)
    for k in range(NG):
        gtk = _unpack_cols(gt_ref[k])
        v1k = v1_ref[k].astype(F32)
        v2k = v2_ref[k].astype(F32)
        wk = w_ref[:, k:k + 1] * winv
        dk = jnp.sum(v2k * gtk, axis=1, keepdims=True)
        t1 = gtk - 2.0 * dk * v2k
        bq = jnp.sum(v1k * t1, axis=1, keepdims=True)
        q_gt = t1 - 2.0 * bq * v1k
        ak = jnp.sum(v1k * hcb, axis=1, keepdims=True)
        t2 = hcb - 2.0 * ak * v1k
        cq = jnp.sum(v2k * t2, axis=1, keepdims=True)
        qt_h = t2 - 2.0 * cq * v2k
        acc = acc + wk * (hcb - q_gt)
        contrib_ref[k] = wk * (gtk - qt_h)
    asum_ref[...] = acc


def _step_call(hc, gt3, v1, v2, w2d, wsum):
    jblk = 128
    grid = (L // jblk,)
    return pl.pallas_call(
        _step_body,
        grid=grid,
        in_specs=[
            pl.BlockSpec((jblk, D), lambda i: (i, 0)),
            pl.BlockSpec((NG, jblk, HD), lambda i: (0, i, 0)),
            pl.BlockSpec((NG, jblk, D), lambda i: (0, i, 0)),
            pl.BlockSpec((NG, jblk, D), lambda i: (0, i, 0)),
            pl.BlockSpec((jblk, 16), lambda i: (i, 0)),
            pl.BlockSpec(memory_space=pltpu.SMEM),
        ],
        out_specs=[
            pl.BlockSpec((jblk, D), lambda i: (i, 0)),
            pl.BlockSpec((NG, jblk, D), lambda i: (0, i, 0)),
        ],
        out_shape=[
            jax.ShapeDtypeStruct((L, D), F32),
            jax.ShapeDtypeStruct((NG, L, D), F32),
        ],
    )(hc, gt3, v1, v2, w2d, wsum)


def _update_body(hc_ref, h0_ref, asum_ref, p_ref, eta_ref, out_ref, outb_ref):
    g = asum_ref[...] + p_ref[0] + p_ref[1] + LAM * (hc_ref[...] - h0_ref[...])
    hn = hc_ref[...] - jnp.abs(eta_ref[0, 0]) * g
    out_ref[...] = hn
    outb_ref[...] = _pack_cols(hn)


def _update_call(hc, h0, asum, p, eta11):
    jblk = 256
    return pl.pallas_call(
        _update_body,
        grid=(L // jblk,),
        in_specs=[
            pl.BlockSpec((jblk, D), lambda i: (i, 0)),
            pl.BlockSpec((jblk, D), lambda i: (i, 0)),
            pl.BlockSpec((jblk, D), lambda i: (i, 0)),
            pl.BlockSpec((2, jblk, D), lambda i: (0, i, 0)),
            pl.BlockSpec(memory_space=pltpu.SMEM),
        ],
        out_specs=[
            pl.BlockSpec((jblk, D), lambda i: (i, 0)),
            pl.BlockSpec((jblk, HD), lambda i: (i, 0)),
        ],
        out_shape=[
            jax.ShapeDtypeStruct((L, D), F32),
            jax.ShapeDtypeStruct((L, HD), jnp.int32),
        ],
    )(hc, h0, asum, p, eta11)


def _pack_call(hp):
    n = hp.shape[0]
    jblk = 256

    def body(h_ref, o_ref):
        o_ref[...] = _pack_cols(h_ref[...])

    return pl.pallas_call(
        body,
        grid=(n // jblk,),
        in_specs=[pl.BlockSpec((jblk, D), lambda i: (i, 0))],
        out_specs=pl.BlockSpec((jblk, HD), lambda i: (i, 0)),
        out_shape=jax.ShapeDtypeStruct((n, HD), jnp.int32),
    )(hp)


def _energy_body(hc_ref, gt_ref, v1_ref, v2_ref, w_ref, wsum_ref, out_ref):
    i = pl.program_id(0)
    hcb = hc_ref[...]
    winv = 1.0 / (2047.0 + wsum_ref[0, 0] + 1e-8)
    nj = jnp.sum(hcb * hcb, axis=1, keepdims=True)
    tot = jnp.zeros((1, 1), F32)
    for k in range(NG):
        gtk = _unpack_cols(gt_ref[k])
        v1k = v1_ref[k].astype(F32)
        v2k = v2_ref[k].astype(F32)
        wk = w_ref[:, k:k + 1] * winv
        dk = jnp.sum(v2k * gtk, axis=1, keepdims=True)
        t1 = gtk - 2.0 * dk * v2k
        bq = jnp.sum(v1k * t1, axis=1, keepdims=True)
        q_gt = t1 - 2.0 * bq * v1k
        hj_qgt = jnp.sum(hcb * q_gt, axis=1, keepdims=True)
        nt = jnp.sum(gtk * gtk, axis=1, keepdims=True)
        tot = tot + jnp.sum(wk * (nj + nt - 2.0 * hj_qgt),
                            axis=0, keepdims=True)

    @pl.when(i == 0)
    def _():
        out_ref[0, 0] = 0.0

    out_ref[0, 0] += tot[0, 0] * (50.0 / (E_TRUE * D))


def _energy_call(hc, gt3, v1, v2, w2d, wsum):
    jblk = 128
    return pl.pallas_call(
        _energy_body,
        grid=(L // jblk,),
        in_specs=[
            pl.BlockSpec((jblk, D), lambda i: (i, 0)),
            pl.BlockSpec((NG, jblk, HD), lambda i: (0, i, 0)),
            pl.BlockSpec((NG, jblk, D), lambda i: (0, i, 0)),
            pl.BlockSpec((NG, jblk, D), lambda i: (0, i, 0)),
            pl.BlockSpec((jblk, 16), lambda i: (i, 0)),
            pl.BlockSpec(memory_space=pltpu.SMEM),
        ],
        out_specs=pl.BlockSpec(memory_space=pltpu.SMEM),
        out_shape=jax.ShapeDtypeStruct((1, 1), F32),
    )(hc, gt3, v1, v2, w2d, wsum)


def _sc_gather(table, gidx, chunk=32, nbuf=4):
    n_rows = gidx.shape[0]
    width = table.shape[1]
    dt = table.dtype
    nw = 32
    per_w = n_rows // nw
    n_chunks = per_w // chunk
    mesh = plsc.VectorSubcoreMesh(core_axis_name="c", subcore_axis_name="s")

    @functools.partial(
        pl.kernel,
        mesh=mesh,
        out_type=jax.ShapeDtypeStruct((n_rows, width), dt),
        scratch_types=(
            [pltpu.VMEM((per_w,), jnp.int32)]
            + [pltpu.VMEM((chunk, width), dt) for _ in range(nbuf)]
            + [pltpu.SemaphoreType.DMA, pltpu.SemaphoreType.DMA]
        ),
    )
    def k(table_hbm, idx_hbm, out_hbm, idx_v, *bufs_sems):
        bufs = bufs_sems[:nbuf]
        semg, semo = bufs_sems[nbuf], bufs_sems[nbuf + 1]
        wid = lax.axis_index("s") * 2 + lax.axis_index("c")
        base = wid * per_w
        pltpu.async_copy(idx_hbm.at[pl.ds(base, per_w)], idx_v, semg).wait()

        @pl.loop(0, n_chunks, step=nbuf)
        def _(ci):
            for b in range(nbuf):
                cj = ci + b

                @pl.when(cj < n_chunks)
                def _():
                    off = base + cj * chunk

                    @pl.when(cj >= nbuf)
                    def _():
                        pltpu.make_async_copy(
                            bufs[b], out_hbm.at[pl.ds(off, chunk)], semo
                        ).wait()

                    pltpu.async_copy(
                        table_hbm.at[idx_v.at[pl.ds(cj * chunk, chunk)]],
                        bufs[b], semg).wait()
                    pltpu.async_copy(bufs[b], out_hbm.at[pl.ds(off, chunk)],
                                     semo)

        for _ in range(min(nbuf, n_chunks)):
            pltpu.make_async_copy(bufs[0], out_hbm.at[pl.ds(base, chunk)],
                                  semo).wait()

    return k(table, gidx)


def _sc_scatter(src, sidx_rep):
    n_rows = sidx_rep.shape[0]
    nsub = 16
    cw = D // nsub
    per_core = n_rows // 2
    chunk = 192
    n_chunks = per_core // chunk
    assert n_chunks % 2 == 0
    mesh = plsc.VectorSubcoreMesh(core_axis_name="c", subcore_axis_name="s")

    @functools.partial(
        pl.kernel,
        mesh=mesh,
        out_type=jax.ShapeDtypeStruct((2, L, D), F32),
        compiler_params=pltpu.CompilerParams(use_tc_tiling_on_sc=False,
                                             needs_layout_passes=False),
        scratch_types=(
            [pltpu.VMEM((chunk, 16), jnp.int32) for _ in range(2)]
            + [pltpu.VMEM((chunk, cw), F32) for _ in range(2)]
            + [pltpu.VMEM((L, cw), F32), pltpu.SemaphoreType.DMA]
        ),
    )
    def k(src_hbm, idx_hbm, out_hbm, idx0, idx1, rows0, rows1, acc_v, semd):
        idxb = (idx0, idx1)
        rowsb = (rows0, rows1)
        core = lax.axis_index("c")
        sid = lax.axis_index("s")
        cbase = sid * cw
        ebase = core * per_core
        col16 = lax.iota(jnp.int32, 16)
        z16 = jnp.zeros((16,), F32)

        @pl.loop(0, L)
        def _(r):
            for c16 in range(0, cw, 16):
                acc_v[r, pl.ds(c16, 16)] = z16

        def fire(cj, b):
            off = ebase + cj * chunk
            pltpu.async_copy(idx_hbm.at[pl.ds(off, chunk)], idxb[b], semd)
            pltpu.async_copy(
                src_hbm.at[pl.ds(off, chunk), pl.ds(cbase, cw)],
                rowsb[b], semd)

        fire(0, 0)
        fire(1, 1)

        @pl.loop(0, n_chunks, step=2)
        def _(ci):
            for b in range(2):
                cj = ci + b

                @pl.when(cj + 2 < n_chunks)
                def _():
                    fire(cj + 2, b)

                pltpu.make_async_copy(idx_hbm.at[pl.ds(ebase, chunk)],
                                      idxb[b], semd).wait()
                pltpu.make_async_copy(
                    src_hbm.at[pl.ds(ebase, chunk), pl.ds(cbase, cw)],
                    rowsb[b], semd).wait()

                @pl.loop(0, chunk)
                def _(r):
                    trow = idxb[b][r]
                    for c16 in range(0, cw, 16):
                        x = rowsb[b][r, pl.ds(c16, 16)]
                        plsc.addupdate_scatter(acc_v, [trow, col16 + c16], x)

        pltpu.sync_copy(acc_v,
                        out_hbm.at[core, pl.ds(0, L), pl.ds(cbase, cw)])

    return k(src, sidx_rep)


def kernel(h, attn, pos_emb, W1, b1, g1, be1, W2, b2, eta):
    h2 = h[0]
    a2 = attn[0]
    idx8, w8, pij8, pji8, wsum = _topk_call(a2)

    j = jnp.arange(L, dtype=jnp.int32)
    tgt = jnp.concatenate([jnp.maximum(j - 1, 0)[None, :], idx8.T], axis=0)
    pijg = jnp.concatenate(
        [jnp.full((1, L), 1023, jnp.int32), pij8.T], axis=0)
    pjig = jnp.concatenate(
        [jnp.full((1, L), 1025, jnp.int32), pji8.T], axis=0)
    w2d = jnp.concatenate(
        [(j >= 1).astype(F32)[:, None], w8, jnp.zeros((L, 16 - 1 - TOPK), F32)],
        axis=1)
    tgt_flat = tgt.reshape(-1)
    gidx = jnp.concatenate(
        [tgt_flat, pijg.reshape(-1) + L, pjig.reshape(-1) + L])

    perm = np.concatenate([np.arange(0, D, 2), np.arange(1, D, 2)])
    inv_perm = np.argsort(perm)

    tblp = jnp.concatenate([h2[:, perm], pos_emb[:, perm]], axis=0)
    tbl_i = _pack_call(tblp)
    g_i = _sc_gather(tbl_i, gidx, chunk=64)
    ght = g_i[0:NE].reshape(NG, L, HD)
    gpij = g_i[NE:2 * NE].reshape(NG, L, HD)
    gpji = g_i[2 * NE:].reshape(NG, L, HD)

    w1a_bf = W1[:D].astype(BF)
    w1b_bf = W1[D:2 * D].astype(BF)
    ha, hb = _pre_call(h2.astype(BF), w1a_bf, w1b_bf)
    v1, v2 = _mlp_call(
        ha, hb, ght, gpij, gpji,
        W1[:D][perm].astype(BF), W1[D:2 * D][perm].astype(BF),
        W1[2 * D:][perm].astype(BF), W2[:, perm].astype(BF),
        b1.reshape(1, HID), g1.reshape(1, HID), be1.reshape(1, HID),
        b2[perm].reshape(1, D))

    eta11 = eta.reshape(1, 1)
    tgt_rep = jnp.broadcast_to(tgt_flat[:, None], (NE, 16))

    hc = h2[:, perm]
    h0p = hc
    hti = _pack_call(hc)
    for _ in range(KSTEPS):
        gt_i = _sc_gather(hti, tgt_flat, chunk=64).reshape(NG, L, HD)
        asum, contrib = _step_call(hc, gt_i, v1, v2, w2d, wsum)
        p = _sc_scatter(contrib.reshape(NE, D), tgt_rep)
        hc, hti = _update_call(hc, h0p, asum, p, eta11)

    gt_i = _sc_gather(hti, tgt_flat, chunk=64).reshape(NG, L, HD)
    energy = _energy_call(hc, gt_i, v1, v2, w2d, wsum)
    return hc[:, inv_perm][None], energy.reshape(())


# speedup vs baseline: 2.4445x; 1.2378x over previous
import functools

import jax
import jax.numpy as jnp
import numpy as np
from jax import lax
from jax.experimental import pallas as pl
from jax.experimental.pallas import tpu as pltpu
from jax.experimental.pallas import tpu_sc as plsc

L = 2048
D = 768
HID = 2 * D
TOPK = 8
NG = TOPK + 1
KSTEPS = 5
LAM = 0.01
E_TRUE = (L - 1) + TOPK * L
NE = NG * L

BF = jnp.bfloat16
F32 = jnp.float32


def _topk_body(attn_ref, idx_ref, w_ref, pij_ref, pji_ref, wsum_ref):
    i = pl.program_id(0)
    rblk = attn_ref.shape[0]
    a = attn_ref[...]
    rows = i * rblk + lax.broadcasted_iota(jnp.int32, (rblk, L), 0)
    cols = lax.broadcasted_iota(jnp.int32, (rblk, L), 1)
    m = jnp.where(cols <= rows - 2, a, -10000.0)
    vals, idxs = [], []
    for _ in range(TOPK):
        mx = jnp.max(m, axis=1, keepdims=True)
        ismax = m == mx
        ik = jnp.min(jnp.where(ismax, cols, L), axis=1, keepdims=True)
        m = jnp.where(cols == ik, -jnp.float32(jnp.inf), m)
        vals.append(mx)
        idxs.append(ik)
    vals = jnp.concatenate(vals, axis=1)
    idx = jnp.concatenate(idxs, axis=1)
    mxv = jnp.max(vals, axis=1, keepdims=True)
    ex = jnp.exp(vals - mxv)
    w8 = ex / jnp.sum(ex, axis=1, keepdims=True)
    rows8 = i * rblk + lax.broadcasted_iota(jnp.int32, (rblk, TOPK), 0)
    rel = idx - rows8
    idx_ref[...] = idx
    w_ref[...] = w8
    pij_ref[...] = jnp.clip(rel + 1024, 0, 2047)
    pji_ref[...] = jnp.clip(-rel + 1024, 0, 2047)

    @pl.when(i == 0)
    def _():
        wsum_ref[0, 0] = 0.0

    wsum_ref[0, 0] += jnp.sum(w8)


def _topk_call(a2):
    rblk = 256
    grid = L // rblk
    return pl.pallas_call(
        _topk_body,
        grid=(grid,),
        in_specs=[pl.BlockSpec((rblk, L), lambda i: (i, 0))],
        out_specs=[
            pl.BlockSpec((rblk, TOPK), lambda i: (i, 0)),
            pl.BlockSpec((rblk, TOPK), lambda i: (i, 0)),
            pl.BlockSpec((rblk, TOPK), lambda i: (i, 0)),
            pl.BlockSpec((rblk, TOPK), lambda i: (i, 0)),
            pl.BlockSpec(memory_space=pltpu.SMEM),
        ],
        out_shape=[
            jax.ShapeDtypeStruct((L, TOPK), jnp.int32),
            jax.ShapeDtypeStruct((L, TOPK), F32),
            jax.ShapeDtypeStruct((L, TOPK), jnp.int32),
            jax.ShapeDtypeStruct((L, TOPK), jnp.int32),
            jax.ShapeDtypeStruct((1, 1), F32),
        ],
    )(a2)


def _pre_body(h_ref, wa_ref, wb_ref, ha_ref, hb_ref):
    x = h_ref[...]
    ha_ref[...] = jnp.dot(x, wa_ref[...], preferred_element_type=F32)
    hb_ref[...] = jnp.dot(x, wb_ref[...], preferred_element_type=F32)


def _pre_call(h_bf, wa_bf, wb_bf):
    cblk = 512
    grid = HID // cblk
    return pl.pallas_call(
        _pre_body,
        grid=(grid,),
        in_specs=[
            pl.BlockSpec((L, D), lambda i: (0, 0)),
            pl.BlockSpec((D, cblk), lambda i: (0, i)),
            pl.BlockSpec((D, cblk), lambda i: (0, i)),
        ],
        out_specs=[
            pl.BlockSpec((L, cblk), lambda i: (0, i)),
            pl.BlockSpec((L, cblk), lambda i: (0, i)),
        ],
        out_shape=[
            jax.ShapeDtypeStruct((L, HID), F32),
            jax.ShapeDtypeStruct((L, HID), F32),
        ],
    )(h_bf, wa_bf, wb_bf)


def _ln_relu(x, g1, be1):
    mu = jnp.mean(x, axis=1, keepdims=True)
    xc = x - mu
    var = jnp.mean(xc * xc, axis=1, keepdims=True)
    ln = xc / jnp.sqrt(var + 1e-5) * g1 + be1
    return jnp.maximum(ln, 0.0)


def _mlp_body(ha_ref, hb_ref, ght_ref, gpij_ref, gpji_ref,
              wa_ref, wb_ref, wc_ref, w2_ref,
              b1_ref, g1_ref, be1_ref, b2_ref,
              v1_ref, v2_ref):
    ght = _unpack_cols(ght_ref[0]).astype(BF)
    b1 = b1_ref[...]
    g1 = g1_ref[...]
    be1 = be1_ref[...]
    b2 = b2_ref[...]
    w2 = w2_ref[...]
    pos_ij = jnp.dot(_unpack_cols(gpij_ref[0]).astype(BF), wc_ref[...],
                     preferred_element_type=F32)
    pos_ji = jnp.dot(_unpack_cols(gpji_ref[0]).astype(BF), wc_ref[...],
                     preferred_element_type=F32)
    pre_ij = (ha_ref[...] + pos_ij + b1
              + jnp.dot(ght, wb_ref[...], preferred_element_type=F32))
    pre_ji = (hb_ref[...] + pos_ji + b1
              + jnp.dot(ght, wa_ref[...], preferred_element_type=F32))
    for pre, out in ((pre_ij, v1_ref), (pre_ji, v2_ref)):
        hdn = _ln_relu(pre, g1, be1).astype(BF)
        v = jnp.dot(hdn, w2, preferred_element_type=F32) + b2
        n = jnp.sqrt(jnp.sum(v * v, axis=1, keepdims=True))
        out[0] = (v / jnp.maximum(n, 1e-8)).astype(BF)


def _mlp_call(ha, hb, ght, gpij, gpji, wa_bf, wb_bf, wc_bf, w2_bf,
              b1, g1, be1, b2):
    jblk = 256
    grid = (L // jblk, NG)
    row = lambda j, k: (j, 0)
    grp = lambda j, k: (k, j, 0)
    fixed = lambda j, k: (0, 0)
    return pl.pallas_call(
        _mlp_body,
        grid=grid,
        in_specs=[
            pl.BlockSpec((jblk, HID), row),
            pl.BlockSpec((jblk, HID), row),
            pl.BlockSpec((1, jblk, HD), grp),
            pl.BlockSpec((1, jblk, HD), grp),
            pl.BlockSpec((1, jblk, HD), grp),
            pl.BlockSpec((D, HID), fixed),
            pl.BlockSpec((D, HID), fixed),
            pl.BlockSpec((D, HID), fixed),
            pl.BlockSpec((HID, D), fixed),
            pl.BlockSpec((1, HID), fixed),
            pl.BlockSpec((1, HID), fixed),
            pl.BlockSpec((1, HID), fixed),
            pl.BlockSpec((1, D), fixed),
        ],
        out_specs=[
            pl.BlockSpec((1, jblk, D), grp),
            pl.BlockSpec((1, jblk, D), grp),
        ],
        out_shape=[
            jax.ShapeDtypeStruct((NG, L, D), BF),
            jax.ShapeDtypeStruct((NG, L, D), BF),
        ],
    )(ha, hb, ght, gpij, gpji, wa_bf, wb_bf, wc_bf, w2_bf, b1, g1, be1, b2)


HD = D // 2


def _unpack_cols(w):
    e = lax.bitcast_convert_type(w << 16, F32)
    o = lax.bitcast_convert_type(w & jnp.int32(-65536), F32)
    return jnp.concatenate([e, o], axis=1)


def _pack_cols(x):
    xb = lax.bitcast_convert_type(x, jnp.int32)
    r = xb + 0x7FFF + ((xb >> 16) & 1)
    b16 = (r >> 16) & 0xFFFF
    return b16[:, :HD] | (b16[:, HD:] << 16)


def _step_body(hc_ref, gt_ref, v1_ref, v2_ref, w_ref, wsum_ref,
               asum_ref, contrib_ref):
    hcb = hc_ref[...]
    winv = 1.0 / (2047.0 + wsum_ref[0, 0] + 1e-8)
    acc = jnp.zeros_like(hcb)
    for k in range(NG):
        gtk = _unpack_cols(gt_ref[k])
        v1k = v1_ref[k].astype(F32)
        v2k = v2_ref[k].astype(F32)
        wk = w_ref[:, k:k + 1] * winv
        dk = jnp.sum(v2k * gtk, axis=1, keepdims=True)
        t1 = gtk - 2.0 * dk * v2k
        bq = jnp.sum(v1k * t1, axis=1, keepdims=True)
        q_gt = t1 - 2.0 * bq * v1k
        ak = jnp.sum(v1k * hcb, axis=1, keepdims=True)
        t2 = hcb - 2.0 * ak * v1k
        cq = jnp.sum(v2k * t2, axis=1, keepdims=True)
        qt_h = t2 - 2.0 * cq * v2k
        acc = acc + wk * (hcb - q_gt)
        contrib_ref[k] = wk * (gtk - qt_h)
    asum_ref[...] = acc


def _step_call(hc, gt3, v1, v2, w2d, wsum):
    jblk = 128
    grid = (L // jblk,)
    return pl.pallas_call(
        _step_body,
        grid=grid,
        in_specs=[
            pl.BlockSpec((jblk, D), lambda i: (i, 0)),
            pl.BlockSpec((NG, jblk, HD), lambda i: (0, i, 0)),
            pl.BlockSpec((NG, jblk, D), lambda i: (0, i, 0)),
            pl.BlockSpec((NG, jblk, D), lambda i: (0, i, 0)),
            pl.BlockSpec((jblk, 16), lambda i: (i, 0)),
            pl.BlockSpec(memory_space=pltpu.SMEM),
        ],
        out_specs=[
            pl.BlockSpec((jblk, D), lambda i: (i, 0)),
            pl.BlockSpec((NG, jblk, D), lambda i: (0, i, 0)),
        ],
        out_shape=[
            jax.ShapeDtypeStruct((L, D), F32),
            jax.ShapeDtypeStruct((NG, L, D), F32),
        ],
    )(hc, gt3, v1, v2, w2d, wsum)


def _update_body(hc_ref, h0_ref, asum_ref, p_ref, eta_ref, out_ref, outb_ref):
    g = asum_ref[...] + p_ref[0] + p_ref[1] + LAM * (hc_ref[...] - h0_ref[...])
    hn = hc_ref[...] - jnp.abs(eta_ref[0, 0]) * g
    out_ref[...] = hn
    outb_ref[...] = _pack_cols(hn)


def _update_call(hc, h0, asum, p, eta11):
    jblk = 256
    return pl.pallas_call(
        _update_body,
        grid=(L // jblk,),
        in_specs=[
            pl.BlockSpec((jblk, D), lambda i: (i, 0)),
            pl.BlockSpec((jblk, D), lambda i: (i, 0)),
            pl.BlockSpec((jblk, D), lambda i: (i, 0)),
            pl.BlockSpec((2, jblk, D), lambda i: (0, i, 0)),
            pl.BlockSpec(memory_space=pltpu.SMEM),
        ],
        out_specs=[
            pl.BlockSpec((jblk, D), lambda i: (i, 0)),
            pl.BlockSpec((jblk, HD), lambda i: (i, 0)),
        ],
        out_shape=[
            jax.ShapeDtypeStruct((L, D), F32),
            jax.ShapeDtypeStruct((L, HD), jnp.int32),
        ],
    )(hc, h0, asum, p, eta11)


def _pack_call(hp):
    n = hp.shape[0]
    jblk = 256

    def body(h_ref, o_ref):
        o_ref[...] = _pack_cols(h_ref[...])

    return pl.pallas_call(
        body,
        grid=(n // jblk,),
        in_specs=[pl.BlockSpec((jblk, D), lambda i: (i, 0))],
        out_specs=pl.BlockSpec((jblk, HD), lambda i: (i, 0)),
        out_shape=jax.ShapeDtypeStruct((n, HD), jnp.int32),
    )(hp)


def _energy_body(hc_ref, gt_ref, v1_ref, v2_ref, w_ref, wsum_ref, out_ref):
    i = pl.program_id(0)
    hcb = hc_ref[...]
    winv = 1.0 / (2047.0 + wsum_ref[0, 0] + 1e-8)
    nj = jnp.sum(hcb * hcb, axis=1, keepdims=True)
    tot = jnp.zeros((1, 1), F32)
    for k in range(NG):
        gtk = _unpack_cols(gt_ref[k])
        v1k = v1_ref[k].astype(F32)
        v2k = v2_ref[k].astype(F32)
        wk = w_ref[:, k:k + 1] * winv
        dk = jnp.sum(v2k * gtk, axis=1, keepdims=True)
        t1 = gtk - 2.0 * dk * v2k
        bq = jnp.sum(v1k * t1, axis=1, keepdims=True)
        q_gt = t1 - 2.0 * bq * v1k
        hj_qgt = jnp.sum(hcb * q_gt, axis=1, keepdims=True)
        nt = jnp.sum(gtk * gtk, axis=1, keepdims=True)
        tot = tot + jnp.sum(wk * (nj + nt - 2.0 * hj_qgt),
                            axis=0, keepdims=True)

    @pl.when(i == 0)
    def _():
        out_ref[0, 0] = 0.0

    out_ref[0, 0] += tot[0, 0] * (50.0 / (E_TRUE * D))


def _energy_call(hc, gt3, v1, v2, w2d, wsum):
    jblk = 128
    return pl.pallas_call(
        _energy_body,
        grid=(L // jblk,),
        in_specs=[
            pl.BlockSpec((jblk, D), lambda i: (i, 0)),
            pl.BlockSpec((NG, jblk, HD), lambda i: (0, i, 0)),
            pl.BlockSpec((NG, jblk, D), lambda i: (0, i, 0)),
            pl.BlockSpec((NG, jblk, D), lambda i: (0, i, 0)),
            pl.BlockSpec((jblk, 16), lambda i: (i, 0)),
            pl.BlockSpec(memory_space=pltpu.SMEM),
        ],
        out_specs=pl.BlockSpec(memory_space=pltpu.SMEM),
        out_shape=jax.ShapeDtypeStruct((1, 1), F32),
    )(hc, gt3, v1, v2, w2d, wsum)


def _sc_gather(table, gidx, chunk=32, nbuf=4):
    n_rows = gidx.shape[0]
    width = table.shape[1]
    dt = table.dtype
    nw = 32
    per_w = n_rows // nw
    n_chunks = per_w // chunk
    mesh = plsc.VectorSubcoreMesh(core_axis_name="c", subcore_axis_name="s")

    @functools.partial(
        pl.kernel,
        mesh=mesh,
        out_type=jax.ShapeDtypeStruct((n_rows, width), dt),
        scratch_types=(
            [pltpu.VMEM((per_w,), jnp.int32)]
            + [pltpu.VMEM((chunk, width), dt) for _ in range(nbuf)]
            + [pltpu.SemaphoreType.DMA, pltpu.SemaphoreType.DMA]
        ),
    )
    def k(table_hbm, idx_hbm, out_hbm, idx_v, *bufs_sems):
        bufs = bufs_sems[:nbuf]
        semg, semo = bufs_sems[nbuf], bufs_sems[nbuf + 1]
        wid = lax.axis_index("s") * 2 + lax.axis_index("c")
        base = wid * per_w
        pltpu.async_copy(idx_hbm.at[pl.ds(base, per_w)], idx_v, semg).wait()

        @pl.loop(0, n_chunks, step=nbuf)
        def _(ci):
            for b in range(nbuf):
                cj = ci + b

                @pl.when(cj < n_chunks)
                def _():
                    off = base + cj * chunk

                    @pl.when(cj >= nbuf)
                    def _():
                        pltpu.make_async_copy(
                            bufs[b], out_hbm.at[pl.ds(off, chunk)], semo
                        ).wait()

                    pltpu.async_copy(
                        table_hbm.at[idx_v.at[pl.ds(cj * chunk, chunk)]],
                        bufs[b], semg).wait()
                    pltpu.async_copy(bufs[b], out_hbm.at[pl.ds(off, chunk)],
                                     semo)

        for _ in range(min(nbuf, n_chunks)):
            pltpu.make_async_copy(bufs[0], out_hbm.at[pl.ds(base, chunk)],
                                  semo).wait()

    return k(table, gidx)


def _sc_scatter(src, sidx_rep):
    n_rows = sidx_rep.shape[0]
    nsub = 16
    cw = D // nsub
    per_core = n_rows // 2
    chunk = 192
    n_chunks = per_core // chunk
    assert n_chunks % 2 == 0
    mesh = plsc.VectorSubcoreMesh(core_axis_name="c", subcore_axis_name="s")

    @functools.partial(
        pl.kernel,
        mesh=mesh,
        out_type=jax.ShapeDtypeStruct((2, L, D), F32),
        compiler_params=pltpu.CompilerParams(use_tc_tiling_on_sc=False,
                                             needs_layout_passes=False),
        scratch_types=(
            [pltpu.VMEM((chunk, 16), jnp.int32) for _ in range(2)]
            + [pltpu.VMEM((chunk, cw), F32) for _ in range(2)]
            + [pltpu.VMEM((L, cw), F32), pltpu.SemaphoreType.DMA]
        ),
    )
    def k(src_hbm, idx_hbm, out_hbm, idx0, idx1, rows0, rows1, acc_v, semd):
        idxb = (idx0, idx1)
        rowsb = (rows0, rows1)
        core = lax.axis_index("c")
        sid = lax.axis_index("s")
        cbase = sid * cw
        ebase = core * per_core
        col16 = lax.iota(jnp.int32, 16)
        z16 = jnp.zeros((16,), F32)

        @pl.loop(0, L)
        def _(r):
            for c16 in range(0, cw, 16):
                acc_v[r, pl.ds(c16, 16)] = z16

        def fire(cj, b):
            off = ebase + cj * chunk
            pltpu.async_copy(idx_hbm.at[pl.ds(off, chunk)], idxb[b], semd)
            pltpu.async_copy(
                src_hbm.at[pl.ds(off, chunk), pl.ds(cbase, cw)],
                rowsb[b], semd)

        fire(0, 0)
        fire(1, 1)

        @pl.loop(0, n_chunks, step=2)
        def _(ci):
            for b in range(2):
                cj = ci + b

                @pl.when(cj + 2 < n_chunks)
                def _():
                    fire(cj + 2, b)

                pltpu.make_async_copy(idx_hbm.at[pl.ds(ebase, chunk)],
                                      idxb[b], semd).wait()
                pltpu.make_async_copy(
                    src_hbm.at[pl.ds(ebase, chunk), pl.ds(cbase, cw)],
                    rowsb[b], semd).wait()

                @plsc.parallel_loop(0, chunk, unroll=8)
                def _(r):
                    trow = idxb[b][r]
                    for c16 in range(0, cw, 16):
                        x = rowsb[b][r, pl.ds(c16, 16)]
                        plsc.addupdate_scatter(acc_v, [trow, col16 + c16], x)

        pltpu.sync_copy(acc_v,
                        out_hbm.at[core, pl.ds(0, L), pl.ds(cbase, cw)])

    return k(src, sidx_rep)


def kernel(h, attn, pos_emb, W1, b1, g1, be1, W2, b2, eta):
    h2 = h[0]
    a2 = attn[0]
    idx8, w8, pij8, pji8, wsum = _topk_call(a2)

    j = jnp.arange(L, dtype=jnp.int32)
    tgt = jnp.concatenate([jnp.maximum(j - 1, 0)[None, :], idx8.T], axis=0)
    pijg = jnp.concatenate(
        [jnp.full((1, L), 1023, jnp.int32), pij8.T], axis=0)
    pjig = jnp.concatenate(
        [jnp.full((1, L), 1025, jnp.int32), pji8.T], axis=0)
    w2d = jnp.concatenate(
        [(j >= 1).astype(F32)[:, None], w8, jnp.zeros((L, 16 - 1 - TOPK), F32)],
        axis=1)
    tgt_flat = tgt.reshape(-1)
    gidx = jnp.concatenate(
        [tgt_flat, pijg.reshape(-1) + L, pjig.reshape(-1) + L])

    perm = np.concatenate([np.arange(0, D, 2), np.arange(1, D, 2)])
    inv_perm = np.argsort(perm)

    tblp = jnp.concatenate([h2[:, perm], pos_emb[:, perm]], axis=0)
    tbl_i = _pack_call(tblp)
    g_i = _sc_gather(tbl_i, gidx, chunk=64)
    ght = g_i[0:NE].reshape(NG, L, HD)
    gpij = g_i[NE:2 * NE].reshape(NG, L, HD)
    gpji = g_i[2 * NE:].reshape(NG, L, HD)

    w1a_bf = W1[:D].astype(BF)
    w1b_bf = W1[D:2 * D].astype(BF)
    ha, hb = _pre_call(h2.astype(BF), w1a_bf, w1b_bf)
    v1, v2 = _mlp_call(
        ha, hb, ght, gpij, gpji,
        W1[:D][perm].astype(BF), W1[D:2 * D][perm].astype(BF),
        W1[2 * D:][perm].astype(BF), W2[:, perm].astype(BF),
        b1.reshape(1, HID), g1.reshape(1, HID), be1.reshape(1, HID),
        b2[perm].reshape(1, D))

    eta11 = eta.reshape(1, 1)
    tgt_rep = jnp.broadcast_to(tgt_flat[:, None], (NE, 16))

    hc = h2[:, perm]
    h0p = hc
    hti = _pack_call(hc)
    for _ in range(KSTEPS):
        gt_i = _sc_gather(hti, tgt_flat, chunk=64).reshape(NG, L, HD)
        asum, contrib = _step_call(hc, gt_i, v1, v2, w2d, wsum)
        p = _sc_scatter(contrib.reshape(NE, D), tgt_rep)
        hc, hti = _update_call(hc, h0p, asum, p, eta11)

    gt_i = _sc_gather(hti, tgt_flat, chunk=64).reshape(NG, L, HD)
    energy = _energy_call(hc, gt_i, v1, v2, w2d, wsum)
    return hc[:, inv_perm][None], energy.reshape(())
